# Initial kernel scaffold; baseline (speedup 1.0000x reference)
import jax
import jax.numpy as jnp
from jax.experimental import pallas as pl


def kernel(input, index, value):
    raise NotImplementedError("write your pallas kernel here")


# speedup vs baseline: 159.0907x; 159.0907x over previous
import functools

import jax
import jax.numpy as jnp
from jax import lax
from jax.experimental import pallas as pl
from jax.experimental.pallas import tpu as pltpu
from jax.experimental.pallas import tpu_sc as plsc

_B, _S, _D, _K = 1024, 200, 128, 32
_NW = 32
_BPW = _B // _NW
_NBUF = 2


def _sc_scatter(inp, idx, val16):
    mesh = plsc.VectorSubcoreMesh(core_axis_name="c", subcore_axis_name="s")

    @functools.partial(
        pl.kernel,
        mesh=mesh,
        compiler_params=pltpu.CompilerParams(needs_layout_passes=False),
        out_type=jax.ShapeDtypeStruct((_B, _S, _D), jnp.float32),
        scratch_types=(
            [pltpu.VMEM((_S, _D), jnp.float32) for _ in range(_NBUF)]
            + [pltpu.VMEM((_S, _K), jnp.int32) for _ in range(_NBUF)]
            + [pltpu.VMEM((16,), jnp.float32)]
            + [pltpu.SemaphoreType.DMA for _ in range(2 * _NBUF)]
        ),
    )
    def k(inp_hbm, idx_hbm, val_hbm, out_hbm, *scratch):
        row_bufs = scratch[0:_NBUF]
        idx_bufs = scratch[_NBUF:2 * _NBUF]
        val_buf = scratch[2 * _NBUF]
        sem_in = scratch[2 * _NBUF + 1:2 * _NBUF + 1 + _NBUF]
        sem_out = scratch[2 * _NBUF + 1 + _NBUF:]

        cid = lax.axis_index("c")
        sid = lax.axis_index("s")
        wid = sid * 2 + cid
        batch0 = wid * _BPW

        pltpu.sync_copy(val_hbm, val_buf)
        vval = val_buf[...]

        def in_copies(ci, b):
            bi = batch0 + ci
            return (
                pltpu.make_async_copy(inp_hbm.at[bi], row_bufs[b], sem_in[b]),
                pltpu.make_async_copy(idx_hbm.at[bi], idx_bufs[b], sem_in[b]),
            )

        def out_copy(ci, b):
            bi = batch0 + ci
            return pltpu.make_async_copy(row_bufs[b], out_hbm.at[bi], sem_out[b])

        def scatter(b):
            rb = row_bufs[b]
            ib = idx_bufs[b]

            @plsc.parallel_loop(0, _S, unroll=8)
            def _row(r):
                iv0 = ib[r, pl.ds(0, 16)]
                iv1 = ib[r, pl.ds(16, 16)]
                rsplat = jnp.full((16,), r, dtype=jnp.int32)
                plsc.store_scatter(rb, [rsplat, iv0], vval)
                plsc.store_scatter(rb, [rsplat, iv1], vval)

        for c in in_copies(0, 0):
            c.start()

        def group(p, carry):
            for b in range(_NBUF):
                ci = p * _NBUF + b
                nb = (b + 1) % _NBUF

                @pl.when(ci + 1 < _BPW)
                def _prefetch():
                    @pl.when(ci >= _NBUF - 1)
                    def _drain():
                        out_copy(ci - (_NBUF - 1), nb).wait()

                    for c in in_copies(ci + 1, nb):
                        c.start()

                for c in in_copies(ci, b):
                    c.wait()
                scatter(b)
                out_copy(ci, b).start()
            return carry

        lax.fori_loop(0, _BPW // _NBUF, group, 0)

        for b in range(_NBUF):
            out_copy(_BPW - _NBUF + b, b).wait()

    return k(inp, idx, val16)


def kernel(input, index, value):
    val16 = jnp.broadcast_to(jnp.asarray(value, input.dtype), (16,))
    return _sc_scatter(input, index, val16)


# speedup vs baseline: 159.5717x; 1.0030x over previous
import functools

import jax
import jax.numpy as jnp
from jax import lax
from jax.experimental import pallas as pl
from jax.experimental.pallas import tpu as pltpu
from jax.experimental.pallas import tpu_sc as plsc

_B, _S, _D, _K = 1024, 200, 128, 32
_N = _B * _S
_NW = 32
_ROWS_PER_W = _N // _NW
_R = 160
_CHUNKS = _ROWS_PER_W // _R
_NBUF = 4


def _sc_scatter(inp2d, idx2d, val16):
    mesh = plsc.VectorSubcoreMesh(core_axis_name="c", subcore_axis_name="s")

    @functools.partial(
        pl.kernel,
        mesh=mesh,
        compiler_params=pltpu.CompilerParams(needs_layout_passes=False),
        out_type=jax.ShapeDtypeStruct((_N, _D), jnp.float32),
        scratch_types=(
            [pltpu.VMEM((_R, _D), jnp.float32) for _ in range(_NBUF)]
            + [pltpu.VMEM((_R * _K,), jnp.int32) for _ in range(_NBUF)]
            + [pltpu.VMEM((16,), jnp.float32)]
            + [pltpu.SemaphoreType.DMA for _ in range(2 * _NBUF)]
        ),
    )
    def k(inp_hbm, idx_hbm, val_hbm, out_hbm, *scratch):
        row_bufs = scratch[0:_NBUF]
        idx_bufs = scratch[_NBUF:2 * _NBUF]
        val_buf = scratch[2 * _NBUF]
        sem_in = scratch[2 * _NBUF + 1:2 * _NBUF + 1 + _NBUF]
        sem_out = scratch[2 * _NBUF + 1 + _NBUF:]

        cid = lax.axis_index("c")
        sid = lax.axis_index("s")
        wid = sid * 2 + cid
        row0 = wid * _ROWS_PER_W

        pltpu.sync_copy(val_hbm, val_buf)
        vval = val_buf[...]

        def in_copies(ci, b):
            base = row0 + ci * _R
            return (
                pltpu.make_async_copy(
                    inp_hbm.at[pl.ds(base, _R)], row_bufs[b], sem_in[b]),
                pltpu.make_async_copy(
                    idx_hbm.at[pl.ds(base * _K, _R * _K)], idx_bufs[b], sem_in[b]),
            )

        def out_copy(ci, b):
            base = row0 + ci * _R
            return pltpu.make_async_copy(
                row_bufs[b], out_hbm.at[pl.ds(base, _R)], sem_out[b])

        def scatter(b):
            rb = row_bufs[b]
            ib = idx_bufs[b]

            @plsc.parallel_loop(0, _R, unroll=8)
            def _row(r):
                iv0 = ib[pl.ds(r * _K, 16)]
                iv1 = ib[pl.ds(r * _K + 16, 16)]
                rsplat = jnp.full((16,), r, dtype=jnp.int32)
                plsc.store_scatter(rb, [rsplat, iv0], vval)
                plsc.store_scatter(rb, [rsplat, iv1], vval)

        for c in in_copies(0, 0):
            c.start()

        def group(p, carry):
            for b in range(_NBUF):
                ci = p * _NBUF + b
                nb = (b + 1) % _NBUF

                @pl.when(ci + 1 < _CHUNKS)
                def _prefetch():
                    @pl.when(ci >= _NBUF - 1)
                    def _drain():
                        out_copy(ci - (_NBUF - 1), nb).wait()

                    for c in in_copies(ci + 1, nb):
                        c.start()

                for c in in_copies(ci, b):
                    c.wait()
                scatter(b)
                out_copy(ci, b).start()
            return carry

        lax.fori_loop(0, _CHUNKS // _NBUF, group, 0)

        for b in range(_NBUF):
            out_copy(_CHUNKS - _NBUF + b, b).wait()

    return k(inp2d, idx2d, val16)


def kernel(input, index, value):
    inp2d = input.reshape(_N, _D)
    idx2d = index.reshape(_N * _K)
    val16 = jnp.broadcast_to(jnp.asarray(value, input.dtype), (16,))
    out = _sc_scatter(inp2d, idx2d, val16)
    return out.reshape(input.shape)


# speedup vs baseline: 255.8165x; 1.6031x over previous
import functools

import jax
import jax.numpy as jnp
from jax import lax
from jax.experimental import pallas as pl
from jax.experimental.pallas import tpu as pltpu
from jax.experimental.pallas import tpu_sc as plsc

_B, _S, _D, _K = 1024, 200, 128, 32
_NW = 32
_BC = 128
_NBC = _B // _BC
_UNITS = _S * _NBC
_UPW = _UNITS // _NW
_NBUF = 5


def _sc_scatter(inp, idx_t, val16):
    mesh = plsc.VectorSubcoreMesh(core_axis_name="c", subcore_axis_name="s")

    @functools.partial(
        pl.kernel,
        mesh=mesh,
        compiler_params=pltpu.CompilerParams(needs_layout_passes=False),
        out_type=jax.ShapeDtypeStruct((_B, _S, _D), jnp.float32),
        scratch_types=(
            [pltpu.VMEM((_BC, _D), jnp.float32) for _ in range(_NBUF)]
            + [pltpu.VMEM((_K, _BC), jnp.int32) for _ in range(_NBUF)]
            + [pltpu.VMEM((16,), jnp.float32)]
            + [pltpu.SemaphoreType.DMA for _ in range(2 * _NBUF)]
        ),
    )
    def k(inp_hbm, idx_hbm, val_hbm, out_hbm, *scratch):
        row_bufs = scratch[0:_NBUF]
        idx_bufs = scratch[_NBUF:2 * _NBUF]
        val_buf = scratch[2 * _NBUF]
        sem_in = scratch[2 * _NBUF + 1:2 * _NBUF + 1 + _NBUF]
        sem_out = scratch[2 * _NBUF + 1 + _NBUF:]

        cid = lax.axis_index("c")
        sid = lax.axis_index("s")
        wid = sid * 2 + cid
        u0 = wid * _UPW

        pltpu.sync_copy(val_hbm, val_buf)
        vval = val_buf[...]
        iota = lax.broadcasted_iota(jnp.int32, (16,), 0)

        def unit_si(u):
            return u // _NBC, (u % _NBC) * _BC

        def in_copies(u, b):
            si, b0 = unit_si(u)
            return (
                pltpu.make_async_copy(
                    inp_hbm.at[pl.ds(b0, _BC), si], row_bufs[b], sem_in[b]),
                pltpu.make_async_copy(
                    idx_hbm.at[si, :, pl.ds(b0, _BC)], idx_bufs[b], sem_in[b]),
            )

        def out_copy(u, b):
            si, b0 = unit_si(u)
            return pltpu.make_async_copy(
                row_bufs[b], out_hbm.at[pl.ds(b0, _BC), si], sem_out[b])

        def scatter(b):
            rb = row_bufs[b]
            ib = idx_bufs[b]

            @plsc.parallel_loop(0, _BC, unroll=8)
            def _row(r):
                rsplat = jnp.full((16,), r, dtype=jnp.int32)
                iv0 = plsc.load_gather(ib, [iota, rsplat])
                iv1 = plsc.load_gather(ib, [iota + 16, rsplat])
                plsc.store_scatter(rb, [rsplat, iv0], vval)
                plsc.store_scatter(rb, [rsplat, iv1], vval)

        for c in in_copies(u0, 0):
            c.start()

        def group(p, carry):
            for b in range(_NBUF):
                j = p * _NBUF + b
                u = u0 + j
                nb = (b + 1) % _NBUF

                @pl.when(j + 1 < _UPW)
                def _prefetch():
                    @pl.when(j >= _NBUF - 1)
                    def _drain():
                        out_copy(u - (_NBUF - 1), nb).wait()

                    for c in in_copies(u + 1, nb):
                        c.start()

                for c in in_copies(u, b):
                    c.wait()
                scatter(b)
                out_copy(u, b).start()
            return carry

        lax.fori_loop(0, _UPW // _NBUF, group, 0)

        for b in range(_NBUF):
            out_copy(u0 + _UPW - _NBUF + b, b).wait()

    return k(inp, idx_t, val16)


def kernel(input, index, value):
    idx_t = jnp.transpose(index, (1, 2, 0))
    val16 = jnp.broadcast_to(jnp.asarray(value, input.dtype), (16,))
    return _sc_scatter(input, idx_t, val16)


# speedup vs baseline: 290.7739x; 1.1367x over previous
import functools

import jax
import jax.numpy as jnp
from jax import lax
from jax.experimental import pallas as pl
from jax.experimental.pallas import tpu as pltpu
from jax.experimental.pallas import tpu_sc as plsc

_B, _S, _D, _K = 1024, 200, 128, 32
_NW = 32
_BC = 128
_NBC = _B // _BC
_UNITS = _S * _NBC
_UPW = _UNITS // _NW
_NBUF = 5


def _sc_scatter(inp, idx_t, val16):
    mesh = plsc.VectorSubcoreMesh(core_axis_name="c", subcore_axis_name="s")

    @functools.partial(
        pl.kernel,
        mesh=mesh,
        compiler_params=pltpu.CompilerParams(needs_layout_passes=False),
        out_type=jax.ShapeDtypeStruct((_B, _S, _D), jnp.float32),
        scratch_types=(
            [pltpu.VMEM((_BC, _D), jnp.float32) for _ in range(_NBUF)]
            + [pltpu.VMEM((_K, _BC), jnp.int32) for _ in range(_NBUF)]
            + [pltpu.VMEM((16,), jnp.float32)]
            + [pltpu.SemaphoreType.DMA for _ in range(2 * _NBUF)]
        ),
    )
    def k(inp_hbm, idx_hbm, val_hbm, out_hbm, *scratch):
        row_bufs = scratch[0:_NBUF]
        idx_bufs = scratch[_NBUF:2 * _NBUF]
        val_buf = scratch[2 * _NBUF]
        sem_in = scratch[2 * _NBUF + 1:2 * _NBUF + 1 + _NBUF]
        sem_out = scratch[2 * _NBUF + 1 + _NBUF:]

        cid = lax.axis_index("c")
        sid = lax.axis_index("s")
        wid = sid * 2 + cid
        u0 = wid * _UPW

        pltpu.sync_copy(val_hbm, val_buf)
        vval = val_buf[...]
        iota = lax.broadcasted_iota(jnp.int32, (16,), 0)

        def unit_si(u):
            return u // _NBC, (u % _NBC) * _BC

        def in_copies(u, b):
            si, b0 = unit_si(u)
            return (
                pltpu.make_async_copy(
                    inp_hbm.at[pl.ds(b0, _BC), si], row_bufs[b], sem_in[b]),
                pltpu.make_async_copy(
                    idx_hbm.at[si, :, pl.ds(b0, _BC)], idx_bufs[b], sem_in[b]),
            )

        def out_copy(u, b):
            si, b0 = unit_si(u)
            return pltpu.make_async_copy(
                row_bufs[b], out_hbm.at[pl.ds(b0, _BC), si], sem_out[b])

        rowvecs = [iota + 16 * g for g in range(_BC // 16)]

        def scatter(b):
            rb = row_bufs[b]
            ib = idx_bufs[b]

            for g in range(_BC // 16):
                rv = rowvecs[g]

                @plsc.parallel_loop(0, _K, unroll=8)
                def _kk(kk, rv=rv):
                    vec = ib[kk, pl.ds(16 * g, 16)]
                    plsc.store_scatter(rb, [rv, vec], vval)

        for c in in_copies(u0, 0):
            c.start()

        def group(p, carry):
            for b in range(_NBUF):
                j = p * _NBUF + b
                u = u0 + j
                nb = (b + 1) % _NBUF

                @pl.when(j + 1 < _UPW)
                def _prefetch():
                    @pl.when(j >= _NBUF - 1)
                    def _drain():
                        out_copy(u - (_NBUF - 1), nb).wait()

                    for c in in_copies(u + 1, nb):
                        c.start()

                for c in in_copies(u, b):
                    c.wait()
                scatter(b)
                out_copy(u, b).start()
            return carry

        lax.fori_loop(0, _UPW // _NBUF, group, 0)

        for b in range(_NBUF):
            out_copy(u0 + _UPW - _NBUF + b, b).wait()

    return k(inp, idx_t, val16)


def kernel(input, index, value):
    idx_t = jnp.transpose(index, (1, 2, 0))
    val16 = jnp.broadcast_to(jnp.asarray(value, input.dtype), (16,))
    return _sc_scatter(input, idx_t, val16)
